# baseline (speedup 1.0000x reference)
import jax
import jax.numpy as jnp
from jax import lax
from jax.experimental import pallas as pl
from jax.experimental.pallas import tpu as pltpu
from jax.experimental.pallas import tpu_sc as plsc

VOCAB = 100000
HIDDEN = 128
MAXLEN = 2048
BATCH = 4
SEQ = 2048

NC = 2
NS = 16
NW = NC * NS
ROWS = BATCH * SEQ
RPW = ROWS // NW
NG = 8
GCHUNK = RPW // NG
NCH = HIDDEN // 16


def _body(x_hbm, segm_hbm, tok_hbm, segtab_hbm, pos_hbm, out_hbm,
          idx_v, segm_v, pos_v, add_v, segtab_v,
          sem_g0, sem_g1, sem_g2, sem_g3, sem_g4, sem_g5, sem_g6, sem_g7,
          sem_in, sem_o):
    sems = (sem_g0, sem_g1, sem_g2, sem_g3, sem_g4, sem_g5, sem_g6, sem_g7)
    wid = lax.axis_index("s") * NC + lax.axis_index("c")
    base = wid * RPW
    pos_base = lax.rem(base, SEQ)

    in_copies = [
        pltpu.async_copy(x_hbm.at[wid], idx_v, sem_in),
        pltpu.async_copy(segm_hbm.at[wid], segm_v, sem_in),
        pltpu.async_copy(segtab_hbm, segtab_v, sem_in),
        pltpu.async_copy(pos_hbm.at[pl.ds(pos_base, RPW)], pos_v, sem_in),
    ]
    for ic in in_copies:
        ic.wait()

    seg0 = [segtab_v[0, pl.ds(c * 16, 16)] for c in range(NCH)]
    diff = [segtab_v[1, pl.ds(c * 16, 16)] - seg0[c] for c in range(NCH)]

    gathers = []
    for j in range(NG):
        @plsc.parallel_loop(j * GCHUNK, (j + 1) * GCHUNK, unroll=2)
        def addend_row(r):
            mv = segm_v[r, :]
            for c in range(NCH):
                sl = pl.ds(c * 16, 16)
                add_v[r, sl] = pos_v[r, sl] + (seg0[c] + mv * diff[c])

        gathers.append(
            pltpu.async_copy(tok_hbm.at[idx_v.at[j]],
                             add_v.at[pl.ds(j * GCHUNK, GCHUNK)], sems[j],
                             add=True))

    out_copies = []
    for j in range(NG):
        gathers[j].wait()
        out_copies.append(
            pltpu.async_copy(add_v.at[pl.ds(j * GCHUNK, GCHUNK)],
                             out_hbm.at[pl.ds(base + j * GCHUNK, GCHUNK)],
                             sem_o))
    for oc in out_copies:
        oc.wait()


@jax.jit
def _run(x3, segm, tok_table, seg_table, pos_table):
    mesh = plsc.VectorSubcoreMesh(core_axis_name="c", subcore_axis_name="s",
                                  num_cores=NC, num_subcores=NS)
    fn = pl.kernel(
        _body,
        out_type=jax.ShapeDtypeStruct((ROWS, HIDDEN), jnp.float32),
        mesh=mesh,
        scratch_types=[
            pltpu.VMEM((NG, GCHUNK), jnp.int32),
            pltpu.VMEM((RPW, 16), jnp.float32),
            pltpu.VMEM((RPW, HIDDEN), jnp.float32),
            pltpu.VMEM((RPW, HIDDEN), jnp.float32),
            pltpu.VMEM((2, HIDDEN), jnp.float32),
            pltpu.SemaphoreType.DMA,
            pltpu.SemaphoreType.DMA,
            pltpu.SemaphoreType.DMA,
            pltpu.SemaphoreType.DMA,
            pltpu.SemaphoreType.DMA,
            pltpu.SemaphoreType.DMA,
            pltpu.SemaphoreType.DMA,
            pltpu.SemaphoreType.DMA,
            pltpu.SemaphoreType.DMA,
            pltpu.SemaphoreType.DMA,
        ],
    )
    return fn(x3, segm, tok_table, seg_table, pos_table)


def kernel(x, segment_ids, tok_table, seg_table, pos_table):
    x3 = x.reshape(NW, NG, GCHUNK).astype(jnp.int32)
    segm = jnp.broadcast_to(
        segment_ids.reshape(NW, RPW, 1).astype(jnp.float32), (NW, RPW, 16))
    out = _run(x3, segm, tok_table, seg_table, pos_table)
    return out.reshape(BATCH, SEQ, HIDDEN)


# speedup vs baseline: 1.0397x; 1.0397x over previous
import jax
import jax.numpy as jnp
from jax import lax
from jax.experimental import pallas as pl
from jax.experimental.pallas import tpu as pltpu
from jax.experimental.pallas import tpu_sc as plsc

VOCAB = 100000
HIDDEN = 128
MAXLEN = 2048
BATCH = 4
SEQ = 2048

NC = 2
NS = 16
NW = NC * NS
ROWS = BATCH * SEQ
PPW = SEQ // NW
RPW = BATCH * PPW
NCH = HIDDEN // 16


def _body(x_hbm, segm_hbm, tok_hbm, segtab_hbm, pos_hbm, out_hbm,
          idx_v, segm_v, pos_v, add_v, segtab_v,
          sem_g0, sem_g1, sem_g2, sem_g3, sem_in, sem_o):
    sems = (sem_g0, sem_g1, sem_g2, sem_g3)
    wid = lax.axis_index("s") * NC + lax.axis_index("c")
    pos_base = wid * PPW

    in_copies = [
        pltpu.async_copy(x_hbm.at[wid], idx_v, sem_in),
        pltpu.async_copy(segm_hbm.at[wid], segm_v, sem_in),
        pltpu.async_copy(segtab_hbm, segtab_v, sem_in),
        pltpu.async_copy(pos_hbm.at[pl.ds(pos_base, PPW)], pos_v, sem_in),
    ]
    for ic in in_copies:
        ic.wait()

    seg0 = [segtab_v[0, pl.ds(c * 16, 16)] for c in range(NCH)]
    diff = [segtab_v[1, pl.ds(c * 16, 16)] - seg0[c] for c in range(NCH)]

    gathers = []
    for b in range(BATCH):
        @plsc.parallel_loop(0, PPW, unroll=2)
        def addend_row(p):
            r = b * PPW + p
            mv = segm_v[r, :]
            for c in range(NCH):
                sl = pl.ds(c * 16, 16)
                add_v[r, sl] = pos_v[p, sl] + (seg0[c] + mv * diff[c])

        gathers.append(
            pltpu.async_copy(tok_hbm.at[idx_v.at[b]],
                             add_v.at[pl.ds(b * PPW, PPW)], sems[b],
                             add=True))

    out_copies = []
    for b in range(BATCH):
        gathers[b].wait()
        out_copies.append(
            pltpu.async_copy(add_v.at[pl.ds(b * PPW, PPW)],
                             out_hbm.at[pl.ds(b * SEQ + pos_base, PPW)],
                             sem_o))
    for oc in out_copies:
        oc.wait()


@jax.jit
def _run(x4, segm, tok_table, seg_table, pos_table):
    mesh = plsc.VectorSubcoreMesh(core_axis_name="c", subcore_axis_name="s",
                                  num_cores=NC, num_subcores=NS)
    fn = pl.kernel(
        _body,
        out_type=jax.ShapeDtypeStruct((ROWS, HIDDEN), jnp.float32),
        mesh=mesh,
        scratch_types=[
            pltpu.VMEM((BATCH, PPW), jnp.int32),
            pltpu.VMEM((RPW, 16), jnp.float32),
            pltpu.VMEM((PPW, HIDDEN), jnp.float32),
            pltpu.VMEM((RPW, HIDDEN), jnp.float32),
            pltpu.VMEM((2, HIDDEN), jnp.float32),
            pltpu.SemaphoreType.DMA,
            pltpu.SemaphoreType.DMA,
            pltpu.SemaphoreType.DMA,
            pltpu.SemaphoreType.DMA,
            pltpu.SemaphoreType.DMA,
            pltpu.SemaphoreType.DMA,
        ],
    )
    return fn(x4, segm, tok_table, seg_table, pos_table)


def kernel(x, segment_ids, tok_table, seg_table, pos_table):
    x4 = x.reshape(BATCH, NW, PPW).transpose(1, 0, 2).astype(jnp.int32)
    segm = jnp.broadcast_to(
        segment_ids.reshape(BATCH, NW, PPW).transpose(1, 0, 2)
        .reshape(NW, RPW, 1).astype(jnp.float32), (NW, RPW, 16))
    out = _run(x4, segm, tok_table, seg_table, pos_table)
    return out.reshape(BATCH, SEQ, HIDDEN)
